# Initial kernel scaffold; baseline (speedup 1.0000x reference)
import jax
import jax.numpy as jnp
from jax.experimental import pallas as pl


def kernel(ori_adj, d_mtr, edge_gradient, adj_with_insert, batch_users_unique, tril_idx0, tril_idx1):
    raise NotImplementedError("write your pallas kernel here")


# speedup vs baseline: 64.9650x; 64.9650x over previous
import functools

import jax
import jax.numpy as jnp
from jax.experimental import pallas as pl


def _fkey(x):
    i = jax.lax.bitcast_convert_type(x, jnp.int32)
    return jnp.where(i >= 0, i, i ^ jnp.int32(0x7FFFFFFF))


def _score_kernel(a_ref, g_ref, gt_ref, awi_ref, dcol_ref, drow_ref,
                  rmrow_ref, rmcol_ref, srm_ref, sin_ref, ksum_ref):
    ti = pl.program_id(0)
    a = a_ref[...]
    rows = ti * a.shape[0] + jax.lax.broadcasted_iota(jnp.int32, a.shape, 0)
    cols = jax.lax.broadcasted_iota(jnp.int32, a.shape, 1)
    lower = rows > cols
    d_i = dcol_ref[...]
    d_j = drow_ref[...]
    f32 = jnp.float32
    bf = lambda x: x.astype(jnp.bfloat16).astype(f32)
    dbi, dbj = bf(d_i), bf(d_j)
    egn = bf(dbi * bf(g_ref[...])) * dbj
    egn_t = bf(dbj * bf(gt_ref[...])) * dbi
    srm = a * (rmrow_ref[...] * egn_t)
    sin = egn * (awi_ref[...] - a)
    srm_ref[...] = jnp.where(lower, srm, jnp.float32(jnp.inf))
    sin_ref[...] = jnp.where(lower, sin, -jnp.float32(jnp.inf))
    part = jnp.sum(a * rmcol_ref[...])

    @pl.when(ti == 0)
    def _init():
        ksum_ref[...] = jnp.zeros((1, 1), jnp.float32)

    ksum_ref[...] = ksum_ref[...] + part


def _select_kernel(scores_ref, k_ref, out_ref, *, negate):
    x = scores_ref[...]
    if negate:
        x = -x
    keys = _fkey(x)
    k = k_ref[0, 0]

    def body(_, carry):
        lo, hi = carry
        mid = (lo & hi) + ((lo ^ hi) >> 1)
        cnt = jnp.sum((keys <= mid).astype(jnp.int32))
        pred = cnt >= k
        return (jnp.where(pred, lo, mid + jnp.int32(1)),
                jnp.where(pred, mid, hi))

    _, hi = jax.lax.fori_loop(
        0, 32, body,
        (jnp.int32(jnp.iinfo(jnp.int32).min), jnp.int32(jnp.iinfo(jnp.int32).max)))
    out_ref[...] = jnp.broadcast_to(hi, (1, 1))


def _assemble_kernel(srm_ref, sin_ref, a_ref, krm_ref, kin_ref, out_ref):
    ti = pl.program_id(0)
    a = a_ref[...]
    rows = ti * a.shape[0] + jax.lax.broadcasted_iota(jnp.int32, a.shape, 0)
    cols = jax.lax.broadcasted_iota(jnp.int32, a.shape, 1)
    lower = rows > cols
    ins = _fkey(-sin_ref[...]) <= kin_ref[0, 0]
    rem = _fkey(srm_ref[...]) <= krm_ref[0, 0]
    v = jnp.where(ins, jnp.float32(1.0),
                  jnp.where(rem, jnp.float32(0.0), a))
    out_ref[...] = jnp.where(lower, v, jnp.float32(0.0))


def kernel(ori_adj, d_mtr, edge_gradient, adj_with_insert, batch_users_unique,
           tril_idx0, tril_idx1):
    n = ori_adj.shape[0]
    b = batch_users_unique.shape[0]
    k_ins = max(int(0.001 * b * (b - 1) / 2), 1)

    d = jnp.diagonal(d_mtr)
    row_mask = jnp.zeros((n,), ori_adj.dtype).at[batch_users_unique].set(1.0)
    gt = edge_gradient.T
    d_col = d.reshape(n, 1)
    d_row = d.reshape(1, n)
    rm_col = row_mask.reshape(n, 1)
    rm_row = row_mask.reshape(1, n)

    tile = 256
    grid = n // tile
    row_spec = pl.BlockSpec((tile, n), lambda i: (i, 0))
    colvec_spec = pl.BlockSpec((tile, 1), lambda i: (i, 0))
    rowvec_spec = pl.BlockSpec((1, n), lambda i: (0, 0))
    scalar_spec = pl.BlockSpec((1, 1), lambda i: (0, 0))
    f32 = jnp.float32

    srm, sin, ksum = pl.pallas_call(
        _score_kernel,
        grid=(grid,),
        in_specs=[row_spec, row_spec, row_spec, row_spec,
                  colvec_spec, rowvec_spec, rowvec_spec, colvec_spec],
        out_specs=[row_spec, row_spec, scalar_spec],
        out_shape=[jax.ShapeDtypeStruct((n, n), f32),
                   jax.ShapeDtypeStruct((n, n), f32),
                   jax.ShapeDtypeStruct((1, 1), f32)],
    )(ori_adj, edge_gradient, gt, adj_with_insert, d_col, d_row, rm_row, rm_col)

    k_rm = jnp.maximum(
        jnp.floor(jnp.float32(0.1) * ksum[0, 0]).astype(jnp.int32), 1
    ).reshape(1, 1)
    k_in = jnp.full((1, 1), k_ins, jnp.int32)

    krm_key = pl.pallas_call(
        functools.partial(_select_kernel, negate=False),
        out_shape=jax.ShapeDtypeStruct((1, 1), jnp.int32),
    )(srm, k_rm)
    kin_key = pl.pallas_call(
        functools.partial(_select_kernel, negate=True),
        out_shape=jax.ShapeDtypeStruct((1, 1), jnp.int32),
    )(sin, k_in)

    v = pl.pallas_call(
        _assemble_kernel,
        grid=(grid,),
        in_specs=[row_spec, row_spec, row_spec, scalar_spec, scalar_spec],
        out_specs=row_spec,
        out_shape=jax.ShapeDtypeStruct((n, n), f32),
    )(srm, sin, ori_adj, krm_key, kin_key)

    return v + v.T
